# baseline (speedup 1.0000x reference)
import jax
import jax.numpy as jnp
from jax import lax
from jax.experimental import pallas as pl
from jax.experimental.pallas import tpu as pltpu
from jax.experimental.pallas import tpu_sc as plsc

N = 10000
E = 320000
IN_DIM = 128
HID = 256
OUT_DIM = 32

NC = 2
NS = 16
K = 400
ZR = 200
WT = 10
DCH = N // WT
DCHZ = 1008

_HIGH = lax.Precision.HIGHEST


def _elu(x):
    return jnp.where(x > 0, x, jnp.exp(jnp.minimum(x, 0.0)) - 1.0)


def _a1_body(x_ref, sum_ref, max_ref):
    i = pl.program_id(0)
    blk = x_ref[...]
    bsum = jnp.sum(blk, axis=0, keepdims=True)
    bmax = jnp.max(blk, axis=0, keepdims=True)

    @pl.when(i == 0)
    def _():
        sum_ref[...] = bsum
        max_ref[...] = bmax

    @pl.when(i > 0)
    def _():
        sum_ref[...] = sum_ref[...] + bsum
        max_ref[...] = jnp.maximum(max_ref[...], bmax)


def _col_reduce(x):
    grid = 10
    bs = N // grid
    return pl.pallas_call(
        _a1_body,
        grid=(grid,),
        in_specs=[pl.BlockSpec((bs, IN_DIM), lambda i: (i, 0))],
        out_specs=[pl.BlockSpec((1, IN_DIM), lambda i: (0, 0))] * 2,
        out_shape=[jax.ShapeDtypeStruct((1, IN_DIM), jnp.float32)] * 2,
    )(x)


def _stab_update(i, a_s, a_d, ss_ref, sd_ref):
    ms = jnp.full((1, IN_DIM), jnp.max(a_s))
    md = jnp.full((1, IN_DIM), jnp.max(a_d))

    @pl.when(i == 0)
    def _():
        ss_ref[...] = ms
        sd_ref[...] = md

    @pl.when(i > 0)
    def _():
        ss_ref[...] = jnp.maximum(ss_ref[...], ms)
        sd_ref[...] = jnp.maximum(sd_ref[...], md)


def _a2_body(x_ref, ps_ref, pm_ref, W1_ref, b1_ref, W2_ref, b2_ref,
             ls_ref, as_ref, ld_ref, ad_ref,
             h1_ref, asum_ref, adum_ref, ss_ref, sd_ref):
    i = pl.program_id(0)
    x = x_ref[...]
    p_avg = ps_ref[...] / N
    p_max = pm_ref[...]

    def mlp(p):
        t = jnp.maximum(jnp.dot(p, W1_ref[...], precision=_HIGH) + b1_ref[...], 0.0)
        return jnp.dot(t, W2_ref[...], precision=_HIGH) + b2_ref[...]

    att = mlp(p_avg) + mlp(p_max)
    g = 1.0 / (1.0 + jnp.exp(-att))
    h1 = 1.5 * (x * g) + x
    h1_ref[...] = h1
    w_s = jnp.dot(ls_ref[...], as_ref[...].T, precision=_HIGH)
    w_d = jnp.dot(ld_ref[...], ad_ref[...].T, precision=_HIGH)
    a_s = jnp.dot(h1, w_s, precision=_HIGH)
    a_d = jnp.dot(h1, w_d, precision=_HIGH)
    asum_ref[...] = a_s
    adum_ref[...] = a_d
    _stab_update(i, a_s, a_d, ss_ref, sd_ref)


def _stage_a(x, W1, b1, W2, b2, lin_src1, att_src1, lin_dst1, att_dst1):
    psum, pmax = _col_reduce(x)
    grid = 5
    bs = N // grid
    full = lambda i: (0, 0)
    return pl.pallas_call(
        _a2_body,
        grid=(grid,),
        in_specs=[
            pl.BlockSpec((bs, IN_DIM), lambda i: (i, 0)),
            pl.BlockSpec((1, IN_DIM), full),
            pl.BlockSpec((1, IN_DIM), full),
            pl.BlockSpec(W1.shape, full),
            pl.BlockSpec((1, W1.shape[1]), full),
            pl.BlockSpec(W2.shape, full),
            pl.BlockSpec((1, IN_DIM), full),
            pl.BlockSpec(lin_src1.shape, full),
            pl.BlockSpec((1, HID), full),
            pl.BlockSpec(lin_dst1.shape, full),
            pl.BlockSpec((1, HID), full),
        ],
        out_specs=[
            pl.BlockSpec((bs, IN_DIM), lambda i: (i, 0)),
            pl.BlockSpec((bs, 1), lambda i: (i, 0)),
            pl.BlockSpec((bs, 1), lambda i: (i, 0)),
            pl.BlockSpec((1, IN_DIM), full),
            pl.BlockSpec((1, IN_DIM), full),
        ],
        out_shape=[
            jax.ShapeDtypeStruct((N, IN_DIM), jnp.float32),
            jax.ShapeDtypeStruct((N, 1), jnp.float32),
            jax.ShapeDtypeStruct((N, 1), jnp.float32),
            jax.ShapeDtypeStruct((1, IN_DIM), jnp.float32),
            jax.ShapeDtypeStruct((1, IN_DIM), jnp.float32),
        ],
    )(x, psum, pmax, W1, b1.reshape(1, -1), W2, b2.reshape(1, -1),
      lin_src1, att_src1.reshape(1, -1), lin_dst1, att_dst1.reshape(1, -1))


def _b_body(acc_ref, den_ref, l1_ref, l2s_ref, a2s_ref, l2d_ref, a2d_ref,
            xs2_ref, as_ref, ad_ref, ss_ref, sd_ref):
    i = pl.program_id(0)
    m = (jnp.concatenate([acc_ref[0], acc_ref[1]], axis=-1)
         / (den_ref[...] + 1e-16))
    c1 = jnp.dot(m, l1_ref[...], precision=_HIGH)
    h = _elu(c1)
    xs2_ref[...] = jnp.dot(h, l2s_ref[...], precision=_HIGH)
    w_s = jnp.dot(l2s_ref[...], a2s_ref[...].T, precision=_HIGH)
    w_d = jnp.dot(l2d_ref[...], a2d_ref[...].T, precision=_HIGH)
    a_s = jnp.dot(h, w_s, precision=_HIGH)
    a_d = jnp.dot(h, w_d, precision=_HIGH)
    as_ref[...] = a_s
    ad_ref[...] = a_d
    _stab_update(i, a_s, a_d, ss_ref, sd_ref)


def _stage_b(acc1, den1, lin_src1, lin_src2, att_src2, lin_dst2, att_dst2):
    grid = 5
    bs = N // grid
    full = lambda i: (0, 0)
    return pl.pallas_call(
        _b_body,
        grid=(grid,),
        in_specs=[
            pl.BlockSpec((NC, bs, IN_DIM // NC), lambda i: (0, i, 0)),
            pl.BlockSpec((bs, 1), lambda i: (i, 0)),
            pl.BlockSpec(lin_src1.shape, full),
            pl.BlockSpec(lin_src2.shape, full),
            pl.BlockSpec((1, OUT_DIM), full),
            pl.BlockSpec(lin_dst2.shape, full),
            pl.BlockSpec((1, OUT_DIM), full),
        ],
        out_specs=[
            pl.BlockSpec((bs, OUT_DIM), lambda i: (i, 0)),
            pl.BlockSpec((bs, 1), lambda i: (i, 0)),
            pl.BlockSpec((bs, 1), lambda i: (i, 0)),
            pl.BlockSpec((1, IN_DIM), full),
            pl.BlockSpec((1, IN_DIM), full),
        ],
        out_shape=[
            jax.ShapeDtypeStruct((N, OUT_DIM), jnp.float32),
            jax.ShapeDtypeStruct((N, 1), jnp.float32),
            jax.ShapeDtypeStruct((N, 1), jnp.float32),
            jax.ShapeDtypeStruct((1, IN_DIM), jnp.float32),
            jax.ShapeDtypeStruct((1, IN_DIM), jnp.float32),
        ],
    )(acc1, den1, lin_src1, lin_src2, att_src2.reshape(1, -1),
      lin_dst2, att_dst2.reshape(1, -1))


def _c_body(acc_ref, den_ref, h2_ref):
    h2_ref[...] = (acc_ref[0] + acc_ref[1]) / (den_ref[0] + den_ref[1] + 1e-16)


def _stage_c(acc2, den2):
    grid = 5
    bs = N // grid
    return pl.pallas_call(
        _c_body,
        grid=(grid,),
        in_specs=[
            pl.BlockSpec((NC, bs, OUT_DIM), lambda i: (0, i, 0)),
            pl.BlockSpec((NC, bs, 1), lambda i: (0, i, 0)),
        ],
        out_specs=[pl.BlockSpec((bs, OUT_DIM), lambda i: (i, 0))],
        out_shape=[jax.ShapeDtypeStruct((N, OUT_DIM), jnp.float32)],
    )(acc2, den2)[0]


def _d_body(acc_ref, den_ref, l2s_ref,
            lp_ref, ap_s_ref, lpd_ref, ap_d_ref,
            ldp_ref, adp_s_ref, ldd_ref, adp_d_ref,
            lm_ref, am_s_ref, lmd_ref, am_d_ref,
            xsp_ref, xsd_ref, xsm_ref,
            asp_ref, adp_ref, asd_ref, add_ref, asm_ref, adm_ref,
            ssp_ref, sdp_ref, ssd_ref, sdd_ref, ssm_ref, sdm_ref):
    i = pl.program_id(0)
    m = (acc_ref[0] + acc_ref[1]) / (den_ref[...] + 1e-16)
    c3 = lax.dot_general(m, l2s_ref[...], (((1,), (1,)), ((), ())),
                         precision=_HIGH)
    h3 = _elu(c3)

    def head(l_ref, a_ref, ld_ref, ad_ref, xs_ref, aso_ref, ado_ref,
             ss_ref, sd_ref):
        xs_ref[...] = jnp.dot(h3, l_ref[...], precision=_HIGH)
        w_s = jnp.dot(l_ref[...], a_ref[...].T, precision=_HIGH)
        w_d = jnp.dot(ld_ref[...], ad_ref[...].T, precision=_HIGH)
        a_s = jnp.dot(h3, w_s, precision=_HIGH)
        a_d = jnp.dot(h3, w_d, precision=_HIGH)
        aso_ref[...] = a_s
        ado_ref[...] = a_d
        _stab_update(i, a_s, a_d, ss_ref, sd_ref)

    head(lp_ref, ap_s_ref, lpd_ref, ap_d_ref, xsp_ref, asp_ref, adp_ref,
         ssp_ref, sdp_ref)
    head(ldp_ref, adp_s_ref, ldd_ref, adp_d_ref, xsd_ref, asd_ref, add_ref,
         ssd_ref, sdd_ref)
    head(lm_ref, am_s_ref, lmd_ref, am_d_ref, xsm_ref, asm_ref, adm_ref,
         ssm_ref, sdm_ref)


def _stage_d(acc3, den1, lin_src2,
             lin_src_pi, att_src_pi, lin_dst_pi, att_dst_pi,
             lin_src_disp, att_src_disp, lin_dst_disp, att_dst_disp,
             lin_src_mean, att_src_mean, lin_dst_mean, att_dst_mean):
    grid = 5
    bs = N // grid
    full = lambda i: (0, 0)
    w_specs = []
    w_args = []
    for lw, aw, lwd, awd in (
            (lin_src_pi, att_src_pi, lin_dst_pi, att_dst_pi),
            (lin_src_disp, att_src_disp, lin_dst_disp, att_dst_disp),
            (lin_src_mean, att_src_mean, lin_dst_mean, att_dst_mean)):
        w_specs += [pl.BlockSpec(lw.shape, full), pl.BlockSpec((1, IN_DIM), full),
                    pl.BlockSpec(lwd.shape, full), pl.BlockSpec((1, IN_DIM), full)]
        w_args += [lw, aw.reshape(1, -1), lwd, awd.reshape(1, -1)]
    xs_spec = pl.BlockSpec((bs, IN_DIM), lambda i: (i, 0))
    av_spec = pl.BlockSpec((bs, 1), lambda i: (i, 0))
    st_spec = pl.BlockSpec((1, IN_DIM), full)
    return pl.pallas_call(
        _d_body,
        grid=(grid,),
        in_specs=[
            pl.BlockSpec((NC, bs, OUT_DIM), lambda i: (0, i, 0)),
            pl.BlockSpec((bs, 1), lambda i: (i, 0)),
            pl.BlockSpec(lin_src2.shape, full),
        ] + w_specs,
        out_specs=[xs_spec] * 3 + [av_spec] * 6 + [st_spec] * 6,
        out_shape=([jax.ShapeDtypeStruct((N, IN_DIM), jnp.float32)] * 3
                   + [jax.ShapeDtypeStruct((N, 1), jnp.float32)] * 6
                   + [jax.ShapeDtypeStruct((1, IN_DIM), jnp.float32)] * 6),
    )(acc3, den1, lin_src2, *w_args)


def _e_body(ap_ref, dp_ref, ad_ref, dd_ref, am_ref, dm_ref, sc_ref,
            pi_ref, disp_ref, mean_ref):
    mp = (jnp.concatenate([ap_ref[0], ap_ref[1]], axis=-1)
          / (dp_ref[...] + 1e-16))
    md = (jnp.concatenate([ad_ref[0], ad_ref[1]], axis=-1)
          / (dd_ref[...] + 1e-16))
    mm = (jnp.concatenate([am_ref[0], am_ref[1]], axis=-1)
          / (dm_ref[...] + 1e-16))
    pi_ref[...] = 1.0 / (1.0 + jnp.exp(-mp))
    sp = jnp.maximum(md, 0.0) + jnp.log1p(jnp.exp(-jnp.abs(md)))
    disp_ref[...] = jnp.clip(sp, 0.0001, 10000.0)
    mean_ref[...] = jnp.clip(jnp.exp(mm), 1e-05, 1000000.0) * sc_ref[...]


def _stage_e(accp, denp, accd, dend, accm, denm, scale):
    grid = 5
    bs = N // grid
    a_spec = pl.BlockSpec((NC, bs, IN_DIM // NC), lambda i: (0, i, 0))
    d_spec = pl.BlockSpec((bs, 1), lambda i: (i, 0))
    o_spec = pl.BlockSpec((bs, IN_DIM), lambda i: (i, 0))
    return pl.pallas_call(
        _e_body,
        grid=(grid,),
        in_specs=[a_spec, d_spec, a_spec, d_spec, a_spec, d_spec,
                  pl.BlockSpec((bs, 1), lambda i: (i, 0))],
        out_specs=[o_spec] * 3,
        out_shape=[jax.ShapeDtypeStruct((N, IN_DIM), jnp.float32)] * 3,
    )(accp, denp, accd, dend, accm, denm, scale)


def _sc_zero_fill(vz, rows, d):
    def zr(i, _):
        for j in range(d // 16):
            vz[i, pl.ds(j * 16, 16)] = jnp.zeros((16,), jnp.float32)
        return 0
    lax.fori_loop(0, rows, zr, 0)


def _sc_pass_att(D, split_cols):
    Dc = D // NC if split_cols else D
    ept = (E // NS) if split_cols else (E // NC // NS)
    nchunk = ept // K
    mesh = plsc.VectorSubcoreMesh(core_axis_name="c", subcore_axis_name="s")

    den_shape = (N,) if split_cols else (NC * N,)

    def body(x_hbm, ei_hbm, as_hbm, ad_hbm, ss_hbm, sd_hbm,
             *refs):
        if split_cols:
            (acc_hbm, den_hbm, ex_hbm,
             vidx, vrows, vex, vas, vad, vss, vsd, vz, vzd,
             sh_acc, sh_den, sem, sem_s) = refs
        else:
            (acc_hbm, den_hbm,
             vidx, vrows, vex, vas, vad, vss, vsd, vz, vzd,
             sh_acc, sh_den, sem, sem_s) = refs
        c = lax.axis_index("c")
        s = lax.axis_index("s")
        _sc_zero_fill(vz, ZR, Dc)

        def zd(i, _):
            vzd[pl.ds(i * 16, 16)] = jnp.zeros((16,), jnp.float32)
            return 0
        lax.fori_loop(0, DCHZ // 16, zd, 0)
        pltpu.sync_copy(as_hbm, vas)
        pltpu.sync_copy(ad_hbm, vad)
        pltpu.sync_copy(ss_hbm.at[0], vss)
        pltpu.sync_copy(sd_hbm.at[0], vsd)
        stab = vss[pl.ds(0, 16)] + vsd[pl.ds(0, 16)]

        @pl.when(s < WT)
        def _():
            for j in range(DCH // ZR):
                pltpu.sync_copy(vz, sh_acc.at[pl.ds(s * DCH + j * ZR, ZR)])
            pltpu.sync_copy(vzd.at[pl.ds(0, DCH)], sh_den.at[pl.ds(s * DCH, DCH)])
        plsc.subcore_barrier()

        if split_cols:
            base = s * ept
            den_on = c == 0
            ex_on = c == 1
        else:
            base = c * (E // NC) + s * ept
            den_on = s >= 0
            ex_on = None

        def table(_):
            return x_hbm.at[c] if split_cols else x_hbm

        pltpu.sync_copy(ei_hbm.at[:, pl.ds(base, K)], vidx.at[0])
        pltpu.async_copy(table(0).at[vidx.at[0, 0]], vrows.at[0], sem)

        def chunk(g, _):
            b = lax.rem(g, 2)
            b2 = 1 - b
            off = base + g * K
            @pl.when(g < nchunk - 1)
            def _():
                pltpu.sync_copy(ei_hbm.at[:, pl.ds(off + K, K)], vidx.at[b2])
            pltpu.make_async_copy(table(0).at[vidx.at[b, 0]], vrows.at[b],
                                  sem).wait()

            @pl.when(g < nchunk - 1)
            def _():
                pltpu.async_copy(table(0).at[vidx.at[b2, 0]], vrows.at[b2], sem)

            for t in range(K // 16):
                s16 = vidx[b, 0, pl.ds(t * 16, 16)]
                d16 = vidx[b, 1, pl.ds(t * 16, 16)]
                z = (plsc.load_gather(vas, [s16])
                     + plsc.load_gather(vad, [d16]))
                e = jnp.where(z > 0, z, 0.2 * z)
                vex[b, pl.ds(t * 16, 16)] = jnp.exp(e - stab)

            def scale(t, _):
                exv = vex[b, pl.ds(t * 16, 16)]
                for l in range(16):
                    a = exv[l]
                    r = t * 16 + l
                    for j in range(Dc // 16):
                        vrows[b, r, pl.ds(j * 16, 16)] = (
                            vrows[b, r, pl.ds(j * 16, 16)] * a)
                return 0
            lax.fori_loop(0, K // 16, scale, 0, unroll=5)

            desc = pltpu.make_async_copy(vrows.at[b], sh_acc.at[vidx.at[b, 1]],
                                         sem_s)
            desc.start(add=True)
            if split_cols:
                @pl.when(ex_on)
                def _():
                    pltpu.sync_copy(vex.at[b], ex_hbm.at[pl.ds(off, K)])
            desc.wait()
            if split_cols:
                @pl.when(den_on)
                def _():
                    pltpu.sync_copy(vex.at[b], sh_den.at[vidx.at[b, 1]],
                                    add=True)
            else:
                pltpu.sync_copy(vex.at[b], sh_den.at[vidx.at[b, 1]], add=True)
            return 0
        lax.fori_loop(0, nchunk, chunk, 0)
        plsc.subcore_barrier()

        @pl.when(s < WT)
        def _():
            for j in range(DCH // ZR):
                r0 = s * DCH + j * ZR
                pltpu.sync_copy(sh_acc.at[pl.ds(r0, ZR)], vz)
                pltpu.sync_copy(vz, acc_hbm.at[c, pl.ds(r0, ZR)])

            @pl.when(den_on)
            def _():
                pltpu.sync_copy(sh_den.at[pl.ds(s * DCH, DCH)],
                                vzd.at[pl.ds(0, DCH)])
                if split_cols:
                    pltpu.sync_copy(vzd.at[pl.ds(0, DCH)],
                                    den_hbm.at[pl.ds(s * DCH, DCH)])
                else:
                    pltpu.sync_copy(vzd.at[pl.ds(0, DCH)],
                                    den_hbm.at[pl.ds(c * N + s * DCH, DCH)])

    out_type = [
        jax.ShapeDtypeStruct((NC, N, Dc), jnp.float32),
        jax.ShapeDtypeStruct(den_shape, jnp.float32),
    ]
    if split_cols:
        out_type.append(jax.ShapeDtypeStruct((E,), jnp.float32))

    return pl.kernel(
        body,
        out_type=out_type,
        mesh=mesh,
        scratch_types=[
            pltpu.VMEM((2, 2, K), jnp.int32),
            pltpu.VMEM((2, K, Dc), jnp.float32),
            pltpu.VMEM((2, K), jnp.float32),
            pltpu.VMEM((N,), jnp.float32),
            pltpu.VMEM((N,), jnp.float32),
            pltpu.VMEM((128,), jnp.float32),
            pltpu.VMEM((128,), jnp.float32),
            pltpu.VMEM((ZR, Dc), jnp.float32),
            pltpu.VMEM((DCHZ,), jnp.float32),
            pltpu.VMEM_SHARED((N, Dc), jnp.float32),
            pltpu.VMEM_SHARED((N,), jnp.float32),
            pltpu.SemaphoreType.DMA,
            pltpu.SemaphoreType.DMA,
        ],
        compiler_params=pltpu.CompilerParams(needs_layout_passes=False,
                                             use_tc_tiling_on_sc=False),
    )


def _sc_pass_tied(D):
    ept = E // NC // NS
    nchunk = ept // K
    mesh = plsc.VectorSubcoreMesh(core_axis_name="c", subcore_axis_name="s")

    def body(x_hbm, ei_hbm, exw_hbm, acc_hbm,
             vidx, vrows, vex, vz, sh_acc, sem, sem_s):
        c = lax.axis_index("c")
        s = lax.axis_index("s")
        _sc_zero_fill(vz, ZR, D)

        @pl.when(s < WT)
        def _():
            for j in range(DCH // ZR):
                pltpu.sync_copy(vz, sh_acc.at[pl.ds(s * DCH + j * ZR, ZR)])
        plsc.subcore_barrier()

        base = c * (E // NC) + s * ept

        pltpu.sync_copy(ei_hbm.at[:, pl.ds(base, K)], vidx.at[0])
        pltpu.sync_copy(exw_hbm.at[pl.ds(base, K)], vex.at[0])
        pltpu.async_copy(x_hbm.at[vidx.at[0, 0]], vrows.at[0], sem)

        def chunk(g, _):
            b = lax.rem(g, 2)
            b2 = 1 - b
            off = base + g * K

            @pl.when(g < nchunk - 1)
            def _():
                pltpu.sync_copy(ei_hbm.at[:, pl.ds(off + K, K)], vidx.at[b2])
                pltpu.sync_copy(exw_hbm.at[pl.ds(off + K, K)], vex.at[b2])
            pltpu.make_async_copy(x_hbm.at[vidx.at[b, 0]], vrows.at[b],
                                  sem).wait()

            @pl.when(g < nchunk - 1)
            def _():
                pltpu.async_copy(x_hbm.at[vidx.at[b2, 0]], vrows.at[b2], sem)

            def scale(t, _):
                exv = vex[b, pl.ds(t * 16, 16)]
                for l in range(16):
                    a = exv[l]
                    r = t * 16 + l
                    for j in range(D // 16):
                        vrows[b, r, pl.ds(j * 16, 16)] = (
                            vrows[b, r, pl.ds(j * 16, 16)] * a)
                return 0
            lax.fori_loop(0, K // 16, scale, 0, unroll=5)
            desc = pltpu.make_async_copy(vrows.at[b], sh_acc.at[vidx.at[b, 1]],
                                         sem_s)
            desc.start(add=True)
            desc.wait()
            return 0
        lax.fori_loop(0, nchunk, chunk, 0)
        plsc.subcore_barrier()

        @pl.when(s < WT)
        def _():
            for j in range(DCH // ZR):
                r0 = s * DCH + j * ZR
                pltpu.sync_copy(sh_acc.at[pl.ds(r0, ZR)], vz)
                pltpu.sync_copy(vz, acc_hbm.at[c, pl.ds(r0, ZR)])

    return pl.kernel(
        body,
        out_type=[jax.ShapeDtypeStruct((NC, N, D), jnp.float32)],
        mesh=mesh,
        scratch_types=[
            pltpu.VMEM((2, 2, K), jnp.int32),
            pltpu.VMEM((2, K, D), jnp.float32),
            pltpu.VMEM((2, K), jnp.float32),
            pltpu.VMEM((ZR, D), jnp.float32),
            pltpu.VMEM_SHARED((N, D), jnp.float32),
            pltpu.SemaphoreType.DMA,
            pltpu.SemaphoreType.DMA,
        ],
        compiler_params=pltpu.CompilerParams(needs_layout_passes=False,
                                             use_tc_tiling_on_sc=False),
    )


def kernel(features, edge_index, scale_factor,
           lin_src1, lin_dst1, att_src1, att_dst1,
           lin_src2, lin_dst2, att_src2, att_dst2,
           lin_src_pi, lin_dst_pi, att_src_pi, att_dst_pi,
           lin_src_disp, lin_dst_disp, att_src_disp, att_dst_disp,
           lin_src_mean, lin_dst_mean, att_src_mean, att_dst_mean,
           W1, b1, W2, b2):
    ei = edge_index

    h1, a_s1, a_d1, ss1, sd1 = _stage_a(
        features, W1, b1, W2, b2, lin_src1, att_src1, lin_dst1, att_dst1)

    def split(v):
        return v.reshape(N, NC, IN_DIM // NC).transpose(1, 0, 2)

    acc1, den1, ex1 = _sc_pass_att(IN_DIM, True)(
        split(h1), ei, a_s1.reshape(N), a_d1.reshape(N), ss1, sd1)
    den1_2 = den1.reshape(N, 1)

    xs2, a_s2, a_d2, ss2, sd2 = _stage_b(
        acc1, den1_2, lin_src1, lin_src2, att_src2, lin_dst2, att_dst2)

    acc2, den2 = _sc_pass_att(OUT_DIM, False)(
        xs2, ei, a_s2.reshape(N), a_d2.reshape(N), ss2, sd2)

    h2 = _stage_c(acc2, den2.reshape(NC, N, 1))

    acc3 = _sc_pass_tied(OUT_DIM)(h2, ei, ex1)[0]

    (xsp, xsd, xsm, asp, adp, asd, add_, asm, adm,
     ssp, sdp, ssd, sdd, ssm, sdm) = _stage_d(
        acc3, den1_2, lin_src2,
        lin_src_pi, att_src_pi, lin_dst_pi, att_dst_pi,
        lin_src_disp, att_src_disp, lin_dst_disp, att_dst_disp,
        lin_src_mean, att_src_mean, lin_dst_mean, att_dst_mean)

    sc128 = _sc_pass_att(IN_DIM, True)

    accp, denp, _ = sc128(split(xsp), ei,
                          asp.reshape(N), adp.reshape(N), ssp, sdp)
    accd, dend, _ = sc128(split(xsd), ei,
                          asd.reshape(N), add_.reshape(N), ssd, sdd)
    accm, denm, _ = sc128(split(xsm), ei,
                          asm.reshape(N), adm.reshape(N), ssm, sdm)

    pi, disp, mean = _stage_e(
        accp, denp.reshape(N, 1), accd, dend.reshape(N, 1),
        accm, denm.reshape(N, 1), scale_factor.reshape(N, 1))

    return (mean, disp, pi, h2)


# speedup vs baseline: 1.8684x; 1.8684x over previous
import jax
import jax.numpy as jnp
from jax import lax
from jax.experimental import pallas as pl
from jax.experimental.pallas import tpu as pltpu
from jax.experimental.pallas import tpu_sc as plsc

N = 10000
E = 320000
IN_DIM = 128
HID = 256
OUT_DIM = 32

NC = 2
NS = 16
K = 400
ZR = 200
WT = 10
DCH = N // WT
DCHZ = 1008

_HIGH = lax.Precision.HIGHEST


def _elu(x):
    return jnp.where(x > 0, x, jnp.exp(jnp.minimum(x, 0.0)) - 1.0)


def _a1_body(x_ref, sum_ref, max_ref):
    i = pl.program_id(0)
    blk = x_ref[...]
    bsum = jnp.sum(blk, axis=0, keepdims=True)
    bmax = jnp.max(blk, axis=0, keepdims=True)

    @pl.when(i == 0)
    def _():
        sum_ref[...] = bsum
        max_ref[...] = bmax

    @pl.when(i > 0)
    def _():
        sum_ref[...] = sum_ref[...] + bsum
        max_ref[...] = jnp.maximum(max_ref[...], bmax)


def _col_reduce(x):
    grid = 10
    bs = N // grid
    return pl.pallas_call(
        _a1_body,
        grid=(grid,),
        in_specs=[pl.BlockSpec((bs, IN_DIM), lambda i: (i, 0))],
        out_specs=[pl.BlockSpec((1, IN_DIM), lambda i: (0, 0))] * 2,
        out_shape=[jax.ShapeDtypeStruct((1, IN_DIM), jnp.float32)] * 2,
    )(x)


def _stab_update(i, a_s, a_d, ss_ref, sd_ref):
    ms = jnp.full((1, IN_DIM), jnp.max(a_s))
    md = jnp.full((1, IN_DIM), jnp.max(a_d))

    @pl.when(i == 0)
    def _():
        ss_ref[...] = ms
        sd_ref[...] = md

    @pl.when(i > 0)
    def _():
        ss_ref[...] = jnp.maximum(ss_ref[...], ms)
        sd_ref[...] = jnp.maximum(sd_ref[...], md)


def _a2_body(x_ref, ps_ref, pm_ref, W1_ref, b1_ref, W2_ref, b2_ref,
             ls_ref, as_ref, ld_ref, ad_ref,
             h1_ref, asum_ref, adum_ref, ss_ref, sd_ref):
    i = pl.program_id(0)
    x = x_ref[...]
    p_avg = ps_ref[...] / N
    p_max = pm_ref[...]

    def mlp(p):
        t = jnp.maximum(jnp.dot(p, W1_ref[...], precision=_HIGH) + b1_ref[...], 0.0)
        return jnp.dot(t, W2_ref[...], precision=_HIGH) + b2_ref[...]

    att = mlp(p_avg) + mlp(p_max)
    g = 1.0 / (1.0 + jnp.exp(-att))
    h1 = 1.5 * (x * g) + x
    h1_ref[...] = h1
    w_s = jnp.dot(ls_ref[...], as_ref[...].T, precision=_HIGH)
    w_d = jnp.dot(ld_ref[...], ad_ref[...].T, precision=_HIGH)
    a_s = jnp.dot(h1, w_s, precision=_HIGH)
    a_d = jnp.dot(h1, w_d, precision=_HIGH)
    asum_ref[...] = a_s
    adum_ref[...] = a_d
    _stab_update(i, a_s, a_d, ss_ref, sd_ref)


def _stage_a(x, W1, b1, W2, b2, lin_src1, att_src1, lin_dst1, att_dst1):
    psum, pmax = _col_reduce(x)
    grid = 5
    bs = N // grid
    full = lambda i: (0, 0)
    return pl.pallas_call(
        _a2_body,
        grid=(grid,),
        in_specs=[
            pl.BlockSpec((bs, IN_DIM), lambda i: (i, 0)),
            pl.BlockSpec((1, IN_DIM), full),
            pl.BlockSpec((1, IN_DIM), full),
            pl.BlockSpec(W1.shape, full),
            pl.BlockSpec((1, W1.shape[1]), full),
            pl.BlockSpec(W2.shape, full),
            pl.BlockSpec((1, IN_DIM), full),
            pl.BlockSpec(lin_src1.shape, full),
            pl.BlockSpec((1, HID), full),
            pl.BlockSpec(lin_dst1.shape, full),
            pl.BlockSpec((1, HID), full),
        ],
        out_specs=[
            pl.BlockSpec((bs, IN_DIM), lambda i: (i, 0)),
            pl.BlockSpec((bs, 1), lambda i: (i, 0)),
            pl.BlockSpec((bs, 1), lambda i: (i, 0)),
            pl.BlockSpec((1, IN_DIM), full),
            pl.BlockSpec((1, IN_DIM), full),
        ],
        out_shape=[
            jax.ShapeDtypeStruct((N, IN_DIM), jnp.float32),
            jax.ShapeDtypeStruct((N, 1), jnp.float32),
            jax.ShapeDtypeStruct((N, 1), jnp.float32),
            jax.ShapeDtypeStruct((1, IN_DIM), jnp.float32),
            jax.ShapeDtypeStruct((1, IN_DIM), jnp.float32),
        ],
    )(x, psum, pmax, W1, b1.reshape(1, -1), W2, b2.reshape(1, -1),
      lin_src1, att_src1.reshape(1, -1), lin_dst1, att_dst1.reshape(1, -1))


def _b_body(acc_ref, den_ref, l1_ref, l2s_ref, a2s_ref, l2d_ref, a2d_ref,
            xs2_ref, as_ref, ad_ref, ss_ref, sd_ref):
    i = pl.program_id(0)
    m = (jnp.concatenate([acc_ref[0], acc_ref[1]], axis=-1)
         / (den_ref[...] + 1e-16))
    c1 = jnp.dot(m, l1_ref[...], precision=_HIGH)
    h = _elu(c1)
    xs2_ref[...] = jnp.dot(h, l2s_ref[...], precision=_HIGH)
    w_s = jnp.dot(l2s_ref[...], a2s_ref[...].T, precision=_HIGH)
    w_d = jnp.dot(l2d_ref[...], a2d_ref[...].T, precision=_HIGH)
    a_s = jnp.dot(h, w_s, precision=_HIGH)
    a_d = jnp.dot(h, w_d, precision=_HIGH)
    as_ref[...] = a_s
    ad_ref[...] = a_d
    _stab_update(i, a_s, a_d, ss_ref, sd_ref)


def _stage_b(acc1, den1, lin_src1, lin_src2, att_src2, lin_dst2, att_dst2):
    grid = 5
    bs = N // grid
    full = lambda i: (0, 0)
    return pl.pallas_call(
        _b_body,
        grid=(grid,),
        in_specs=[
            pl.BlockSpec((NC, bs, IN_DIM // NC), lambda i: (0, i, 0)),
            pl.BlockSpec((bs, 1), lambda i: (i, 0)),
            pl.BlockSpec(lin_src1.shape, full),
            pl.BlockSpec(lin_src2.shape, full),
            pl.BlockSpec((1, OUT_DIM), full),
            pl.BlockSpec(lin_dst2.shape, full),
            pl.BlockSpec((1, OUT_DIM), full),
        ],
        out_specs=[
            pl.BlockSpec((bs, OUT_DIM), lambda i: (i, 0)),
            pl.BlockSpec((bs, 1), lambda i: (i, 0)),
            pl.BlockSpec((bs, 1), lambda i: (i, 0)),
            pl.BlockSpec((1, IN_DIM), full),
            pl.BlockSpec((1, IN_DIM), full),
        ],
        out_shape=[
            jax.ShapeDtypeStruct((N, OUT_DIM), jnp.float32),
            jax.ShapeDtypeStruct((N, 1), jnp.float32),
            jax.ShapeDtypeStruct((N, 1), jnp.float32),
            jax.ShapeDtypeStruct((1, IN_DIM), jnp.float32),
            jax.ShapeDtypeStruct((1, IN_DIM), jnp.float32),
        ],
    )(acc1, den1, lin_src1, lin_src2, att_src2.reshape(1, -1),
      lin_dst2, att_dst2.reshape(1, -1))


def _c_body(acc_ref, den_ref, h2_ref):
    h2_ref[...] = (acc_ref[0] + acc_ref[1]) / (den_ref[0] + den_ref[1] + 1e-16)


def _stage_c(acc2, den2):
    grid = 5
    bs = N // grid
    return pl.pallas_call(
        _c_body,
        grid=(grid,),
        in_specs=[
            pl.BlockSpec((NC, bs, OUT_DIM), lambda i: (0, i, 0)),
            pl.BlockSpec((NC, bs, 1), lambda i: (0, i, 0)),
        ],
        out_specs=[pl.BlockSpec((bs, OUT_DIM), lambda i: (i, 0))],
        out_shape=[jax.ShapeDtypeStruct((N, OUT_DIM), jnp.float32)],
    )(acc2, den2)[0]


def _d_body(acc_ref, den_ref, l2s_ref,
            lp_ref, ap_s_ref, lpd_ref, ap_d_ref,
            ldp_ref, adp_s_ref, ldd_ref, adp_d_ref,
            lm_ref, am_s_ref, lmd_ref, am_d_ref,
            xsp_ref, xsd_ref, xsm_ref,
            asp_ref, adp_ref, asd_ref, add_ref, asm_ref, adm_ref,
            ssp_ref, sdp_ref, ssd_ref, sdd_ref, ssm_ref, sdm_ref):
    i = pl.program_id(0)
    m = (acc_ref[0] + acc_ref[1]) / (den_ref[...] + 1e-16)
    c3 = lax.dot_general(m, l2s_ref[...], (((1,), (1,)), ((), ())),
                         precision=_HIGH)
    h3 = _elu(c3)

    def head(l_ref, a_ref, ld_ref, ad_ref, xs_ref, aso_ref, ado_ref,
             ss_ref, sd_ref):
        xs_ref[...] = jnp.dot(h3, l_ref[...], precision=_HIGH)
        w_s = jnp.dot(l_ref[...], a_ref[...].T, precision=_HIGH)
        w_d = jnp.dot(ld_ref[...], ad_ref[...].T, precision=_HIGH)
        a_s = jnp.dot(h3, w_s, precision=_HIGH)
        a_d = jnp.dot(h3, w_d, precision=_HIGH)
        aso_ref[...] = a_s
        ado_ref[...] = a_d
        _stab_update(i, a_s, a_d, ss_ref, sd_ref)

    head(lp_ref, ap_s_ref, lpd_ref, ap_d_ref, xsp_ref, asp_ref, adp_ref,
         ssp_ref, sdp_ref)
    head(ldp_ref, adp_s_ref, ldd_ref, adp_d_ref, xsd_ref, asd_ref, add_ref,
         ssd_ref, sdd_ref)
    head(lm_ref, am_s_ref, lmd_ref, am_d_ref, xsm_ref, asm_ref, adm_ref,
         ssm_ref, sdm_ref)


def _stage_d(acc3, den1, lin_src2,
             lin_src_pi, att_src_pi, lin_dst_pi, att_dst_pi,
             lin_src_disp, att_src_disp, lin_dst_disp, att_dst_disp,
             lin_src_mean, att_src_mean, lin_dst_mean, att_dst_mean):
    grid = 5
    bs = N // grid
    full = lambda i: (0, 0)
    w_specs = []
    w_args = []
    for lw, aw, lwd, awd in (
            (lin_src_pi, att_src_pi, lin_dst_pi, att_dst_pi),
            (lin_src_disp, att_src_disp, lin_dst_disp, att_dst_disp),
            (lin_src_mean, att_src_mean, lin_dst_mean, att_dst_mean)):
        w_specs += [pl.BlockSpec(lw.shape, full), pl.BlockSpec((1, IN_DIM), full),
                    pl.BlockSpec(lwd.shape, full), pl.BlockSpec((1, IN_DIM), full)]
        w_args += [lw, aw.reshape(1, -1), lwd, awd.reshape(1, -1)]
    xs_spec = pl.BlockSpec((bs, IN_DIM), lambda i: (i, 0))
    av_spec = pl.BlockSpec((bs, 1), lambda i: (i, 0))
    st_spec = pl.BlockSpec((1, IN_DIM), full)
    return pl.pallas_call(
        _d_body,
        grid=(grid,),
        in_specs=[
            pl.BlockSpec((NC, bs, OUT_DIM), lambda i: (0, i, 0)),
            pl.BlockSpec((bs, 1), lambda i: (i, 0)),
            pl.BlockSpec(lin_src2.shape, full),
        ] + w_specs,
        out_specs=[xs_spec] * 3 + [av_spec] * 6 + [st_spec] * 6,
        out_shape=([jax.ShapeDtypeStruct((N, IN_DIM), jnp.float32)] * 3
                   + [jax.ShapeDtypeStruct((N, 1), jnp.float32)] * 6
                   + [jax.ShapeDtypeStruct((1, IN_DIM), jnp.float32)] * 6),
    )(acc3, den1, lin_src2, *w_args)


def _e_body(ap_ref, dp_ref, ad_ref, dd_ref, am_ref, dm_ref, sc_ref,
            pi_ref, disp_ref, mean_ref):
    mp = (jnp.concatenate([ap_ref[0], ap_ref[1]], axis=-1)
          / (dp_ref[...] + 1e-16))
    md = (jnp.concatenate([ad_ref[0], ad_ref[1]], axis=-1)
          / (dd_ref[...] + 1e-16))
    mm = (jnp.concatenate([am_ref[0], am_ref[1]], axis=-1)
          / (dm_ref[...] + 1e-16))
    pi_ref[...] = 1.0 / (1.0 + jnp.exp(-mp))
    sp = jnp.maximum(md, 0.0) + jnp.log1p(jnp.exp(-jnp.abs(md)))
    disp_ref[...] = jnp.clip(sp, 0.0001, 10000.0)
    mean_ref[...] = jnp.clip(jnp.exp(mm), 1e-05, 1000000.0) * sc_ref[...]


def _stage_e(accp, denp, accd, dend, accm, denm, scale):
    grid = 5
    bs = N // grid
    a_spec = pl.BlockSpec((NC, bs, IN_DIM // NC), lambda i: (0, i, 0))
    d_spec = pl.BlockSpec((bs, 1), lambda i: (i, 0))
    o_spec = pl.BlockSpec((bs, IN_DIM), lambda i: (i, 0))
    return pl.pallas_call(
        _e_body,
        grid=(grid,),
        in_specs=[a_spec, d_spec, a_spec, d_spec, a_spec, d_spec,
                  pl.BlockSpec((bs, 1), lambda i: (i, 0))],
        out_specs=[o_spec] * 3,
        out_shape=[jax.ShapeDtypeStruct((N, IN_DIM), jnp.float32)] * 3,
    )(accp, denp, accd, dend, accm, denm, scale)


def _sc_zero_fill(vz, rows, d):
    def zr(i, _):
        for j in range(d // 16):
            vz[i, pl.ds(j * 16, 16)] = jnp.zeros((16,), jnp.float32)
        return 0
    lax.fori_loop(0, rows, zr, 0)


def _sc_pass_att(D, split_cols):
    Dc = D // NC if split_cols else D
    ept = (E // NS) if split_cols else (E // NC // NS)
    K = 400 if split_cols else 1000
    nchunk = ept // K
    mesh = plsc.VectorSubcoreMesh(core_axis_name="c", subcore_axis_name="s")

    den_shape = (N,) if split_cols else (NC * N,)

    def body(x_hbm, ei_hbm, as_hbm, ad_hbm, ss_hbm, sd_hbm,
             *refs):
        if split_cols:
            (acc_hbm, den_hbm, ex_hbm,
             vidx, vrows, vex, vas, vad, vss, vsd, vz, vzd,
             sh_acc, sh_den, sem, sem_s) = refs
        else:
            (acc_hbm, den_hbm,
             vidx, vrows, vex, vas, vad, vss, vsd, vz, vzd,
             sh_acc, sh_den, sem, sem_s) = refs
        c = lax.axis_index("c")
        s = lax.axis_index("s")
        _sc_zero_fill(vz, ZR, Dc)

        def zd(i, _):
            vzd[pl.ds(i * 16, 16)] = jnp.zeros((16,), jnp.float32)
            return 0
        lax.fori_loop(0, DCHZ // 16, zd, 0)
        pltpu.sync_copy(as_hbm, vas)
        pltpu.sync_copy(ad_hbm, vad)
        pltpu.sync_copy(ss_hbm.at[0], vss)
        pltpu.sync_copy(sd_hbm.at[0], vsd)
        stab = vss[pl.ds(0, 16)] + vsd[pl.ds(0, 16)]

        @pl.when(s < WT)
        def _():
            for j in range(DCH // ZR):
                pltpu.sync_copy(vz, sh_acc.at[pl.ds(s * DCH + j * ZR, ZR)])
            pltpu.sync_copy(vzd.at[pl.ds(0, DCH)], sh_den.at[pl.ds(s * DCH, DCH)])
        plsc.subcore_barrier()

        if split_cols:
            base = s * ept
            den_on = c == 0
            ex_on = c == 1
        else:
            base = c * (E // NC) + s * ept
            den_on = s >= 0
            ex_on = None

        def table(_):
            return x_hbm.at[c] if split_cols else x_hbm

        pltpu.sync_copy(ei_hbm.at[:, pl.ds(base, K)], vidx.at[0])
        pltpu.async_copy(table(0).at[vidx.at[0, 0]], vrows.at[0], sem)

        def chunk(g, _):
            b = lax.rem(g, 2)
            b2 = 1 - b
            off = base + g * K
            @pl.when(g < nchunk - 1)
            def _():
                pltpu.sync_copy(ei_hbm.at[:, pl.ds(off + K, K)], vidx.at[b2])
            pltpu.make_async_copy(table(0).at[vidx.at[b, 0]], vrows.at[b],
                                  sem).wait()

            @pl.when(g < nchunk - 1)
            def _():
                pltpu.async_copy(table(0).at[vidx.at[b2, 0]], vrows.at[b2], sem)

            for t in range(K // 16):
                s16 = vidx[b, 0, pl.ds(t * 16, 16)]
                d16 = vidx[b, 1, pl.ds(t * 16, 16)]
                z = (plsc.load_gather(vas, [s16])
                     + plsc.load_gather(vad, [d16]))
                e = jnp.where(z > 0, z, 0.2 * z)
                vex[b, pl.ds(t * 16, 16)] = jnp.exp(e - stab)

            def scale(t, _):
                exv = vex[b, pl.ds(t * 16, 16)]
                for l in range(16):
                    a = exv[l]
                    r = t * 16 + l
                    for j in range(Dc // 16):
                        vrows[b, r, pl.ds(j * 16, 16)] = (
                            vrows[b, r, pl.ds(j * 16, 16)] * a)
                return 0
            lax.fori_loop(0, K // 16, scale, 0)

            desc = pltpu.make_async_copy(vrows.at[b], sh_acc.at[vidx.at[b, 1]],
                                         sem_s)
            desc.start(add=True)
            if split_cols:
                @pl.when(ex_on)
                def _():
                    pltpu.sync_copy(vex.at[b], ex_hbm.at[pl.ds(off, K)])
            desc.wait()
            if split_cols:
                @pl.when(den_on)
                def _():
                    pltpu.sync_copy(vex.at[b], sh_den.at[vidx.at[b, 1]],
                                    add=True)
            else:
                pltpu.sync_copy(vex.at[b], sh_den.at[vidx.at[b, 1]], add=True)
            return 0
        lax.fori_loop(0, nchunk, chunk, 0)
        plsc.subcore_barrier()

        @pl.when(s < WT)
        def _():
            for j in range(DCH // ZR):
                r0 = s * DCH + j * ZR
                pltpu.sync_copy(sh_acc.at[pl.ds(r0, ZR)], vz)
                pltpu.sync_copy(vz, acc_hbm.at[c, pl.ds(r0, ZR)])

            @pl.when(den_on)
            def _():
                pltpu.sync_copy(sh_den.at[pl.ds(s * DCH, DCH)],
                                vzd.at[pl.ds(0, DCH)])
                if split_cols:
                    pltpu.sync_copy(vzd.at[pl.ds(0, DCH)],
                                    den_hbm.at[pl.ds(s * DCH, DCH)])
                else:
                    pltpu.sync_copy(vzd.at[pl.ds(0, DCH)],
                                    den_hbm.at[pl.ds(c * N + s * DCH, DCH)])

    out_type = [
        jax.ShapeDtypeStruct((NC, N, Dc), jnp.float32),
        jax.ShapeDtypeStruct(den_shape, jnp.float32),
    ]
    if split_cols:
        out_type.append(jax.ShapeDtypeStruct((E,), jnp.float32))

    return pl.kernel(
        body,
        out_type=out_type,
        mesh=mesh,
        scratch_types=[
            pltpu.VMEM((2, 2, K), jnp.int32),
            pltpu.VMEM((2, K, Dc), jnp.float32),
            pltpu.VMEM((2, K), jnp.float32),
            pltpu.VMEM((N,), jnp.float32),
            pltpu.VMEM((N,), jnp.float32),
            pltpu.VMEM((128,), jnp.float32),
            pltpu.VMEM((128,), jnp.float32),
            pltpu.VMEM((ZR, Dc), jnp.float32),
            pltpu.VMEM((DCHZ,), jnp.float32),
            pltpu.VMEM_SHARED((N, Dc), jnp.float32),
            pltpu.VMEM_SHARED((N,), jnp.float32),
            pltpu.SemaphoreType.DMA,
            pltpu.SemaphoreType.DMA,
        ],
        compiler_params=pltpu.CompilerParams(needs_layout_passes=False,
                                             use_tc_tiling_on_sc=False),
    )


def _sc_pass_tied(D):
    ept = E // NC // NS
    K = 1000
    nchunk = ept // K
    mesh = plsc.VectorSubcoreMesh(core_axis_name="c", subcore_axis_name="s")

    def body(x_hbm, ei_hbm, exw_hbm, acc_hbm,
             vidx, vrows, vex, vz, sh_acc, sem, sem_s):
        c = lax.axis_index("c")
        s = lax.axis_index("s")
        _sc_zero_fill(vz, ZR, D)

        @pl.when(s < WT)
        def _():
            for j in range(DCH // ZR):
                pltpu.sync_copy(vz, sh_acc.at[pl.ds(s * DCH + j * ZR, ZR)])
        plsc.subcore_barrier()

        base = c * (E // NC) + s * ept

        pltpu.sync_copy(ei_hbm.at[:, pl.ds(base, K)], vidx.at[0])
        pltpu.sync_copy(exw_hbm.at[pl.ds(base, K)], vex.at[0])
        pltpu.async_copy(x_hbm.at[vidx.at[0, 0]], vrows.at[0], sem)

        def chunk(g, _):
            b = lax.rem(g, 2)
            b2 = 1 - b
            off = base + g * K

            @pl.when(g < nchunk - 1)
            def _():
                pltpu.sync_copy(ei_hbm.at[:, pl.ds(off + K, K)], vidx.at[b2])
                pltpu.sync_copy(exw_hbm.at[pl.ds(off + K, K)], vex.at[b2])
            pltpu.make_async_copy(x_hbm.at[vidx.at[b, 0]], vrows.at[b],
                                  sem).wait()

            @pl.when(g < nchunk - 1)
            def _():
                pltpu.async_copy(x_hbm.at[vidx.at[b2, 0]], vrows.at[b2], sem)

            def scale(t, _):
                exv = vex[b, pl.ds(t * 16, 16)]
                for l in range(16):
                    a = exv[l]
                    r = t * 16 + l
                    for j in range(D // 16):
                        vrows[b, r, pl.ds(j * 16, 16)] = (
                            vrows[b, r, pl.ds(j * 16, 16)] * a)
                return 0
            lax.fori_loop(0, K // 16, scale, 0)
            desc = pltpu.make_async_copy(vrows.at[b], sh_acc.at[vidx.at[b, 1]],
                                         sem_s)
            desc.start(add=True)
            desc.wait()
            return 0
        lax.fori_loop(0, nchunk, chunk, 0)
        plsc.subcore_barrier()

        @pl.when(s < WT)
        def _():
            for j in range(DCH // ZR):
                r0 = s * DCH + j * ZR
                pltpu.sync_copy(sh_acc.at[pl.ds(r0, ZR)], vz)
                pltpu.sync_copy(vz, acc_hbm.at[c, pl.ds(r0, ZR)])

    return pl.kernel(
        body,
        out_type=[jax.ShapeDtypeStruct((NC, N, D), jnp.float32)],
        mesh=mesh,
        scratch_types=[
            pltpu.VMEM((2, 2, K), jnp.int32),
            pltpu.VMEM((2, K, D), jnp.float32),
            pltpu.VMEM((2, K), jnp.float32),
            pltpu.VMEM((ZR, D), jnp.float32),
            pltpu.VMEM_SHARED((N, D), jnp.float32),
            pltpu.SemaphoreType.DMA,
            pltpu.SemaphoreType.DMA,
        ],
        compiler_params=pltpu.CompilerParams(needs_layout_passes=False,
                                             use_tc_tiling_on_sc=False),
    )


def kernel(features, edge_index, scale_factor,
           lin_src1, lin_dst1, att_src1, att_dst1,
           lin_src2, lin_dst2, att_src2, att_dst2,
           lin_src_pi, lin_dst_pi, att_src_pi, att_dst_pi,
           lin_src_disp, lin_dst_disp, att_src_disp, att_dst_disp,
           lin_src_mean, lin_dst_mean, att_src_mean, att_dst_mean,
           W1, b1, W2, b2):
    ei = edge_index

    h1, a_s1, a_d1, ss1, sd1 = _stage_a(
        features, W1, b1, W2, b2, lin_src1, att_src1, lin_dst1, att_dst1)

    def split(v):
        return v.reshape(N, NC, IN_DIM // NC).transpose(1, 0, 2)

    acc1, den1, ex1 = _sc_pass_att(IN_DIM, True)(
        split(h1), ei, a_s1.reshape(N), a_d1.reshape(N), ss1, sd1)
    den1_2 = den1.reshape(N, 1)

    xs2, a_s2, a_d2, ss2, sd2 = _stage_b(
        acc1, den1_2, lin_src1, lin_src2, att_src2, lin_dst2, att_dst2)

    acc2, den2 = _sc_pass_att(OUT_DIM, False)(
        xs2, ei, a_s2.reshape(N), a_d2.reshape(N), ss2, sd2)

    h2 = _stage_c(acc2, den2.reshape(NC, N, 1))

    acc3 = _sc_pass_tied(OUT_DIM)(h2, ei, ex1)[0]

    (xsp, xsd, xsm, asp, adp, asd, add_, asm, adm,
     ssp, sdp, ssd, sdd, ssm, sdm) = _stage_d(
        acc3, den1_2, lin_src2,
        lin_src_pi, att_src_pi, lin_dst_pi, att_dst_pi,
        lin_src_disp, att_src_disp, lin_dst_disp, att_dst_disp,
        lin_src_mean, att_src_mean, lin_dst_mean, att_dst_mean)

    sc128 = _sc_pass_att(IN_DIM, True)

    accp, denp, _ = sc128(split(xsp), ei,
                          asp.reshape(N), adp.reshape(N), ssp, sdp)
    accd, dend, _ = sc128(split(xsd), ei,
                          asd.reshape(N), add_.reshape(N), ssd, sdd)
    accm, denm, _ = sc128(split(xsm), ei,
                          asm.reshape(N), adm.reshape(N), ssm, sdm)

    pi, disp, mean = _stage_e(
        accp, denp.reshape(N, 1), accd, dend.reshape(N, 1),
        accm, denm.reshape(N, 1), scale_factor.reshape(N, 1))

    return (mean, disp, pi, h2)
